# Initial kernel scaffold; baseline (speedup 1.0000x reference)
import jax
import jax.numpy as jnp
from jax.experimental import pallas as pl


def kernel(x, Wg, W1, W2, W3):
    raise NotImplementedError("write your pallas kernel here")


# speedup vs baseline: 1.5003x; 1.5003x over previous
import jax
import jax.numpy as jnp
from jax.experimental import pallas as pl
from jax.experimental.pallas import tpu as pltpu

_B, _S, _H, _D, _E = 1, 2048, 1024, 3584, 8
_DC = 512
_ND = _D // _DC


def _router_gates(x_f32, wg_f32):
    lg = jax.lax.dot_general(
        x_f32, wg_f32, (((1,), (1,)), ((), ())),
        preferred_element_type=jnp.float32)
    iota = jax.lax.broadcasted_iota(jnp.int32, lg.shape, 1)
    i1 = jnp.argmax(lg, axis=1)[:, None]
    oh1 = iota == i1
    m1 = jnp.max(lg, axis=1, keepdims=True)
    lg2 = jnp.where(oh1, -jnp.inf, lg)
    i2 = jnp.argmax(lg2, axis=1)[:, None]
    m2 = jnp.max(lg2, axis=1, keepdims=True)
    p2 = jnp.exp(m2 - m1)
    denom = 1.0 + p2
    return jnp.where(oh1, 1.0 / denom, 0.0) + jnp.where(iota == i2, p2 / denom, 0.0)


def _moe_body(x_ref, wg_ref, w1_ref, w2_ref, w3_ref, out_ref, xbf_ref, gate_ref):
    e = pl.program_id(0)
    d = pl.program_id(1)

    @pl.when((e == 0) & (d == 0))
    def _init():
        xbf_ref[...] = x_ref[...].astype(jnp.bfloat16)
        out_ref[...] = jnp.zeros_like(out_ref)
        gate_ref[...] = _router_gates(x_ref[...], wg_ref[...])

    w1 = w1_ref[0].astype(jnp.bfloat16)
    w3 = w3_ref[0].astype(jnp.bfloat16)
    w2 = w2_ref[0].astype(jnp.bfloat16)
    xb = xbf_ref[...]
    a1 = jax.lax.dot_general(xb, w1, (((1,), (1,)), ((), ())),
                             preferred_element_type=jnp.float32)
    a3 = jax.lax.dot_general(xb, w3, (((1,), (1,)), ((), ())),
                             preferred_element_type=jnp.float32)
    h = (a1 * jax.nn.sigmoid(a1) * a3).astype(jnp.bfloat16)
    y = jax.lax.dot_general(h, w2, (((1,), (1,)), ((), ())),
                            preferred_element_type=jnp.float32)
    iota8 = jax.lax.broadcasted_iota(jnp.int32, gate_ref.shape, 1)
    g_col = jnp.sum(gate_ref[...] * (iota8 == e).astype(jnp.float32),
                    axis=1, keepdims=True)
    out_ref[...] += g_col * y


def kernel(x, Wg, W1, W2, W3):
    T = _S
    xf = x.reshape(T, _H)
    out = pl.pallas_call(
        _moe_body,
        grid=(_E, _ND),
        in_specs=[
            pl.BlockSpec((T, _H), lambda e, d: (0, 0)),
            pl.BlockSpec((_E, _H), lambda e, d: (0, 0)),
            pl.BlockSpec((1, _DC, _H), lambda e, d: (e, d, 0)),
            pl.BlockSpec((1, _H, _DC), lambda e, d: (e, 0, d)),
            pl.BlockSpec((1, _DC, _H), lambda e, d: (e, d, 0)),
        ],
        out_specs=pl.BlockSpec((T, _H), lambda e, d: (0, 0)),
        out_shape=jax.ShapeDtypeStruct((T, _H), jnp.float32),
        scratch_shapes=[
            pltpu.VMEM((T, _H), jnp.bfloat16),
            pltpu.VMEM((T, _E), jnp.float32),
        ],
        compiler_params=pltpu.CompilerParams(
            dimension_semantics=("arbitrary", "arbitrary"),
        ),
    )(xf, Wg, W1, W2, W3)
    return out.reshape(_B, _S, _H)


# speedup vs baseline: 1.5097x; 1.0063x over previous
import jax
import jax.numpy as jnp
from jax.experimental import pallas as pl
from jax.experimental.pallas import tpu as pltpu

_B, _S, _H, _D, _E = 1, 2048, 1024, 3584, 8
_DC = 512
_ND = _D // _DC


def _router_gates(x_f32, wg_f32):
    lg = jax.lax.dot_general(
        x_f32, wg_f32, (((1,), (1,)), ((), ())),
        preferred_element_type=jnp.float32)
    iota = jax.lax.broadcasted_iota(jnp.int32, lg.shape, 1)
    i1 = jnp.argmax(lg, axis=1)[:, None]
    oh1 = iota == i1
    m1 = jnp.max(lg, axis=1, keepdims=True)
    lg2 = jnp.where(oh1, -jnp.inf, lg)
    i2 = jnp.argmax(lg2, axis=1)[:, None]
    m2 = jnp.max(lg2, axis=1, keepdims=True)
    p2 = jnp.exp(m2 - m1)
    denom = 1.0 + p2
    return jnp.where(oh1, 1.0 / denom, 0.0) + jnp.where(iota == i2, p2 / denom, 0.0)


def _moe_body(x_ref, wg_ref, w1_ref, w2_ref, w3_ref, out_ref, gate_ref):
    e = pl.program_id(0)
    d = pl.program_id(1)

    @pl.when((e == 0) & (d == 0))
    def _init():
        out_ref[...] = jnp.zeros_like(out_ref)
        gate_ref[...] = _router_gates(x_ref[...], wg_ref[...])

    w1 = w1_ref[0]
    w3 = w3_ref[0]
    w2 = w2_ref[0]
    xb = x_ref[...]
    a1 = jax.lax.dot_general(xb, w1, (((1,), (1,)), ((), ())),
                             preferred_element_type=jnp.float32)
    a3 = jax.lax.dot_general(xb, w3, (((1,), (1,)), ((), ())),
                             preferred_element_type=jnp.float32)
    h = a1 * jax.nn.sigmoid(a1) * a3
    y = jax.lax.dot_general(h, w2, (((1,), (1,)), ((), ())),
                            preferred_element_type=jnp.float32)
    iota8 = jax.lax.broadcasted_iota(jnp.int32, gate_ref.shape, 1)
    g_col = jnp.sum(gate_ref[...] * (iota8 == e).astype(jnp.float32),
                    axis=1, keepdims=True)
    out_ref[...] += g_col * y


def kernel(x, Wg, W1, W2, W3):
    T = _S
    xf = x.reshape(T, _H)
    out = pl.pallas_call(
        _moe_body,
        grid=(_E, _ND),
        in_specs=[
            pl.BlockSpec((T, _H), lambda e, d: (0, 0)),
            pl.BlockSpec((_E, _H), lambda e, d: (0, 0)),
            pl.BlockSpec((1, _DC, _H), lambda e, d: (e, d, 0)),
            pl.BlockSpec((1, _H, _DC), lambda e, d: (e, 0, d)),
            pl.BlockSpec((1, _DC, _H), lambda e, d: (e, d, 0)),
        ],
        out_specs=pl.BlockSpec((T, _H), lambda e, d: (0, 0)),
        out_shape=jax.ShapeDtypeStruct((T, _H), jnp.float32),
        scratch_shapes=[
            pltpu.VMEM((T, _E), jnp.float32),
        ],
        compiler_params=pltpu.CompilerParams(
            dimension_semantics=("arbitrary", "arbitrary"),
        ),
    )(xf, Wg, W1, W2, W3)
    return out.reshape(_B, _S, _H)


# speedup vs baseline: 1.5993x; 1.0593x over previous
import functools

import jax
import jax.numpy as jnp
from jax import lax
from jax.experimental import pallas as pl
from jax.experimental.pallas import tpu as pltpu
from jax.experimental.pallas import tpu_sc as plsc

_B, _S, _H, _D, _E = 1, 2048, 1024, 3584, 8
_T = _B * _S
_NP = 2 * _T
_TILE = 256
_NT = _NP // _TILE + _E
_NSLOT = _NT * _TILE
_DC1 = 896
_ND1 = _D // _DC1
_NW = 32
_PPW = _NP // _NW
_CH = 32
_NJ = _PPW // _CH
_TPW = _T // _NW
_CCH = 16
_NCJ = _TPW // _CCH


def _plan_body(x_ref, wg_ref, pos_ref, g16_ref, te_ref, m_ref, rank_ref):
    lg = lax.dot_general(x_ref[...], wg_ref[...], (((1,), (1,)), ((), ())),
                         preferred_element_type=jnp.float32)
    iota = lax.broadcasted_iota(jnp.int32, lg.shape, 1)
    i1 = jnp.argmax(lg, axis=1)[:, None]
    oh1 = iota == i1
    m1 = jnp.max(lg, axis=1, keepdims=True)
    lg2 = jnp.where(oh1, -jnp.inf, lg)
    i2 = jnp.argmax(lg2, axis=1)[:, None]
    oh2 = iota == i2
    m2 = jnp.max(lg2, axis=1, keepdims=True)
    p2 = jnp.exp(m2 - m1)
    denom = 1.0 + p2
    w1n = 1.0 / denom
    w2n = p2 / denom
    m_ref[...] = jnp.concatenate(
        [oh1.astype(jnp.float32), oh2.astype(jnp.float32)], axis=0)
    ones16 = jnp.ones((1, 128), jnp.float32)
    g16_ref[...] = jnp.concatenate([w1n * ones16, w2n * ones16], axis=0)

    r_iota = lax.broadcasted_iota(jnp.int32, (128, 128), 0)
    c_iota = lax.broadcasted_iota(jnp.int32, (128, 128), 1)
    tstrict = (r_iota > c_iota).astype(jnp.float32)

    def chunk(c, carry):
        mc = m_ref[pl.ds(c * 128, 128), :]
        prefix = lax.dot_general(tstrict, mc, (((1,), (0,)), ((), ())),
                                 preferred_element_type=jnp.float32)
        rank_ref[pl.ds(c * 128, 128), :] = prefix + carry
        return carry + jnp.sum(mc, axis=0, keepdims=True)

    counts = lax.fori_loop(0, _NP // 128, chunk,
                           jnp.zeros((1, _E), jnp.float32))
    pc = jnp.floor((counts + float(_TILE - 1)) / float(_TILE)) * float(_TILE)
    er = lax.broadcasted_iota(jnp.int32, (_E, _E), 0)
    ec = lax.broadcasted_iota(jnp.int32, (_E, _E), 1)
    ustrict = (er < ec).astype(jnp.float32)
    off = lax.dot_general(pc, ustrict, (((1,), (0,)), ((), ())),
                          preferred_element_type=jnp.float32)
    cum_end = (off + pc).astype(jnp.int32)
    posm = rank_ref[...] + off
    pos = jnp.sum(posm * m_ref[...], axis=1, keepdims=True)
    pos_ref[...] = pos.astype(jnp.int32)
    sb = lax.broadcasted_iota(jnp.int32, (_NT, _E), 0) * _TILE
    te = jnp.sum((sb >= cum_end).astype(jnp.int32), axis=1)
    te_ref[...] = jnp.minimum(te, _E - 1).reshape(1, _NT)


def _plan(xf, Wg):
    return pl.pallas_call(
        _plan_body,
        grid=(1,),
        in_specs=[pl.BlockSpec((_T, _H), lambda i: (0, 0)),
                  pl.BlockSpec((_E, _H), lambda i: (0, 0))],
        out_specs=[pl.BlockSpec((_NP, 1), lambda i: (0, 0)),
                   pl.BlockSpec((_NP, 128), lambda i: (0, 0)),
                   pl.BlockSpec((1, _NT), lambda i: (0, 0))],
        out_shape=[jax.ShapeDtypeStruct((_NP, 1), jnp.int32),
                   jax.ShapeDtypeStruct((_NP, 128), jnp.float32),
                   jax.ShapeDtypeStruct((1, _NT), jnp.int32)],
        scratch_shapes=[pltpu.VMEM((_NP, _E), jnp.float32),
                        pltpu.VMEM((_NP, _E), jnp.float32)],
    )(xf, Wg)


def _sc_dispatch(xf, pos3, g16):
    mesh = plsc.VectorSubcoreMesh(core_axis_name="c", subcore_axis_name="s")

    @functools.partial(
        pl.kernel, mesh=mesh,
        out_type=[jax.ShapeDtypeStruct((_NSLOT, _H), jnp.float32),
                  jax.ShapeDtypeStruct((_NSLOT, 128), jnp.float32)],
        scratch_types=[pltpu.VMEM((_NJ, _CH), jnp.int32),
                       pltpu.VMEM((_CH, _H), jnp.float32),
                       pltpu.VMEM((_CH, 128), jnp.float32),
                       pltpu.SemaphoreType.DMA,
                       pltpu.SemaphoreType.DMA],
    )
    def k(x_hbm, pos_hbm, g_hbm, xs_hbm, gs_hbm, posv, xbuf, gbuf, sem, sem2):
        wid = lax.axis_index("s") * 2 + lax.axis_index("c")
        pltpu.sync_copy(pos_hbm.at[wid], posv)
        for j in range(_NJ):
            base = wid * _PPW + j * _CH
            toff = lax.rem(base, _T)
            pltpu.sync_copy(x_hbm.at[pl.ds(toff, _CH)], xbuf)
            cp = pltpu.async_copy(xbuf, xs_hbm.at[posv.at[j]], sem)
            pltpu.sync_copy(g_hbm.at[pl.ds(base, _CH)], gbuf)
            cp2 = pltpu.async_copy(gbuf, gs_hbm.at[posv.at[j]], sem2)
            cp.wait()
            cp2.wait()

    return k(xf, pos3, g16)


def _c1_body(te_ref, xs_ref, w1_ref, w3_ref, h_ref):
    a1 = lax.dot_general(xs_ref[...], w1_ref[0], (((1,), (1,)), ((), ())),
                         preferred_element_type=jnp.float32)
    a3 = lax.dot_general(xs_ref[...], w3_ref[0], (((1,), (1,)), ((), ())),
                         preferred_element_type=jnp.float32)
    h_ref[...] = a1 * jax.nn.sigmoid(a1) * a3


def _c1(tef, xs, W1, W3):
    grid_spec = pltpu.PrefetchScalarGridSpec(
        num_scalar_prefetch=1,
        grid=(_ND1, _NT),
        in_specs=[
            pl.BlockSpec((_TILE, _H), lambda d, g, te: (g, 0)),
            pl.BlockSpec((1, _DC1, _H), lambda d, g, te: (te[g], d, 0)),
            pl.BlockSpec((1, _DC1, _H), lambda d, g, te: (te[g], d, 0)),
        ],
        out_specs=pl.BlockSpec((_TILE, _DC1), lambda d, g, te: (g, d)),
    )
    return pl.pallas_call(
        _c1_body,
        grid_spec=grid_spec,
        out_shape=jax.ShapeDtypeStruct((_NSLOT, _D), jnp.float32),
        compiler_params=pltpu.CompilerParams(
            dimension_semantics=("arbitrary", "arbitrary")),
    )(tef, xs, W1, W3)


def _c2_body(te_ref, h_ref, w2_ref, gs_ref, ys_ref):
    y = lax.dot_general(h_ref[...], w2_ref[0], (((1,), (1,)), ((), ())),
                        preferred_element_type=jnp.float32)
    ys_ref[...] = y * gs_ref[:, 0:1]


def _c2(tef, h, W2, gs16):
    grid_spec = pltpu.PrefetchScalarGridSpec(
        num_scalar_prefetch=1,
        grid=(_NT,),
        in_specs=[
            pl.BlockSpec((_TILE, _D), lambda g, te: (g, 0)),
            pl.BlockSpec((1, _H, _D), lambda g, te: (te[g], 0, 0)),
            pl.BlockSpec((_TILE, 128), lambda g, te: (g, 0)),
        ],
        out_specs=pl.BlockSpec((_TILE, _H), lambda g, te: (g, 0)),
    )
    return pl.pallas_call(
        _c2_body,
        grid_spec=grid_spec,
        out_shape=jax.ShapeDtypeStruct((_NSLOT, _H), jnp.float32),
        compiler_params=pltpu.CompilerParams(
            dimension_semantics=("arbitrary",)),
    )(tef, h, W2, gs16)


def _sc_combine(ys, posf):
    mesh = plsc.VectorSubcoreMesh(core_axis_name="c", subcore_axis_name="s")

    @functools.partial(
        pl.kernel, mesh=mesh,
        out_type=jax.ShapeDtypeStruct((_T, _H), jnp.float32),
        scratch_types=[pltpu.VMEM((_CCH,), jnp.int32),
                       pltpu.VMEM((_CCH,), jnp.int32),
                       pltpu.VMEM((_CCH, _H), jnp.float32),
                       pltpu.VMEM((_CCH, _H), jnp.float32),
                       pltpu.SemaphoreType.DMA,
                       pltpu.SemaphoreType.DMA],
    )
    def k(ys_hbm, pos_hbm, out_hbm, idx0, idx1, buf0, buf1, sem, sem2):
        wid = lax.axis_index("s") * 2 + lax.axis_index("c")
        for j in range(_NCJ):
            base = wid * _TPW + j * _CCH
            pltpu.sync_copy(pos_hbm.at[pl.ds(base, _CCH)], idx0)
            pltpu.sync_copy(pos_hbm.at[pl.ds(_T + base, _CCH)], idx1)
            cp0 = pltpu.async_copy(ys_hbm.at[idx0], buf0, sem)
            cp1 = pltpu.async_copy(ys_hbm.at[idx1], buf1, sem2)
            cp0.wait()
            cp1.wait()

            def cbody(ci, _):
                col = ci * 16
                for r in range(_CCH):
                    buf0[r, pl.ds(col, 16)] = (buf0[r, pl.ds(col, 16)]
                                               + buf1[r, pl.ds(col, 16)])
                return 0

            lax.fori_loop(0, _H // 16, cbody, 0)
            pltpu.sync_copy(buf0, out_hbm.at[pl.ds(base, _CCH)])

    return k(ys, posf)


def kernel(x, Wg, W1, W2, W3):
    xf = x.reshape(_T, _H)
    pos, g16, te = _plan(xf, Wg)
    pos3 = pos.reshape(_NW, _NJ, _CH)
    posf = pos.reshape(_NP)
    tef = te.reshape(_NT)
    xs, gs16 = _sc_dispatch(xf, pos3, g16)
    h = _c1(tef, xs, W1, W3)
    ys = _c2(tef, h, W2, gs16)
    out = _sc_combine(ys, posf)
    return out.reshape(_B, _S, _H)
